# baseline (speedup 1.0000x reference)
import functools

import jax
import jax.numpy as jnp
from jax import lax
from jax.experimental import pallas as pl
from jax.experimental.pallas import tpu as pltpu
from jax.experimental.pallas import tpu_sc as plsc

VOCAB = 100000
D = 768
B = 4
S = 2048
N = B * S
NC = 2
NS = 16
NW = NC * NS
TPW = N // NW
CH = 64
NCH = TPW // CH
LANES = 16
VEC = D // LANES


def _positional_encoding(max_seq, d_model):
    pos = jnp.arange(max_seq, dtype=jnp.float32)[:, None]
    i = jnp.arange(0, d_model, 2, dtype=jnp.float32)[None, :]
    denom = jnp.power(10000.0, i / d_model)
    pe = jnp.stack([jnp.sin(pos / denom), jnp.cos(pos / denom)], axis=2)
    return pe.reshape(max_seq, d_model)


def _body(table, tokens, pe, out, idx_v, rows_v, pe_v, sem_g, sem_p):
    wid = lax.axis_index("s") * NC + lax.axis_index("c")
    base = wid * TPW
    pltpu.sync_copy(tokens.at[pl.ds(base, TPW)], idx_v)
    s0 = lax.rem(base, S)
    for c in range(NCH):
        cb = c * CH
        g = pltpu.async_copy(table.at[idx_v.at[pl.ds(cb, CH)]], rows_v, sem_g)
        p = pltpu.async_copy(pe.at[pl.ds(s0 + cb, CH)], pe_v, sem_p)
        g.wait()
        p.wait()

        def add_row(t, carry):
            for j in range(VEC):
                sl = (t, pl.ds(j * LANES, LANES))
                rows_v[sl] = rows_v[sl] + pe_v[sl]
            return carry

        lax.fori_loop(0, CH, add_row, 0)
        pltpu.sync_copy(rows_v, out.at[pl.ds(base + cb, CH)])


@jax.jit
def kernel(tokens, table):
    pe = _positional_encoding(S, D)
    tok = tokens.reshape(N).astype(jnp.int32)
    mesh = plsc.VectorSubcoreMesh(core_axis_name="c", subcore_axis_name="s")
    f = pl.kernel(
        _body,
        out_type=jax.ShapeDtypeStruct((N, D), jnp.float32),
        mesh=mesh,
        scratch_types=[
            pltpu.VMEM((TPW,), jnp.int32),
            pltpu.VMEM((CH, D), jnp.float32),
            pltpu.VMEM((CH, D), jnp.float32),
            pltpu.SemaphoreType.DMA,
            pltpu.SemaphoreType.DMA,
        ],
    )
    out = f(table, tok, pe)
    return out.reshape(B, S, D)


# speedup vs baseline: 1.1279x; 1.1279x over previous
import functools

import jax
import jax.numpy as jnp
from jax import lax
from jax.experimental import pallas as pl
from jax.experimental.pallas import tpu as pltpu
from jax.experimental.pallas import tpu_sc as plsc

VOCAB = 100000
D = 768
B = 4
S = 2048
N = B * S
NC = 2
NS = 16
NW = NC * NS
TPW = N // NW
CH = 32
NCH = TPW // CH
LANES = 16
VEC = D // LANES


def _positional_encoding(max_seq, d_model):
    pos = jnp.arange(max_seq, dtype=jnp.float32)[:, None]
    i = jnp.arange(0, d_model, 2, dtype=jnp.float32)[None, :]
    denom = jnp.power(10000.0, i / d_model)
    pe = jnp.stack([jnp.sin(pos / denom), jnp.cos(pos / denom)], axis=2)
    return pe.reshape(max_seq, d_model)


def _body(table, tokens, pe, out, idx_v,
          rows0, rows1, pe0, pe1,
          sg0, sg1, sp0, sp1):
    rows = (rows0, rows1)
    pes = (pe0, pe1)
    sgs = (sg0, sg1)
    sps = (sp0, sp1)
    wid = lax.axis_index("s") * NC + lax.axis_index("c")
    base = wid * TPW
    pltpu.sync_copy(tokens.at[pl.ds(base, TPW)], idx_v)
    s0 = lax.rem(base, S)

    def start(c):
        i = c % 2
        cb = c * CH
        pltpu.async_copy(table.at[idx_v.at[pl.ds(cb, CH)]], rows[i], sgs[i])
        pltpu.async_copy(pe.at[pl.ds(s0 + cb, CH)], pes[i], sps[i])

    start(0)
    start(1)
    for c in range(NCH):
        i = c % 2
        cb = c * CH
        pltpu.make_async_copy(table.at[idx_v.at[pl.ds(cb, CH)]],
                              rows[i], sgs[i]).wait()
        pltpu.make_async_copy(pe.at[pl.ds(s0 + cb, CH)], pes[i],
                              sps[i]).wait()

        def add_row(t, carry):
            for j in range(VEC):
                sl = (t, pl.ds(j * LANES, LANES))
                rows[i][sl] = rows[i][sl] + pes[i][sl]
            return carry

        lax.fori_loop(0, CH, add_row, 0)
        pltpu.sync_copy(rows[i], out.at[pl.ds(base + cb, CH)])
        if c + 2 < NCH:
            start(c + 2)


@jax.jit
def kernel(tokens, table):
    pe = _positional_encoding(S, D)
    tok = tokens.reshape(N).astype(jnp.int32)
    mesh = plsc.VectorSubcoreMesh(core_axis_name="c", subcore_axis_name="s")
    f = pl.kernel(
        _body,
        out_type=jax.ShapeDtypeStruct((N, D), jnp.float32),
        mesh=mesh,
        scratch_types=[
            pltpu.VMEM((TPW,), jnp.int32),
            pltpu.VMEM((CH, D), jnp.float32),
            pltpu.VMEM((CH, D), jnp.float32),
            pltpu.VMEM((CH, D), jnp.float32),
            pltpu.VMEM((CH, D), jnp.float32),
            pltpu.SemaphoreType.DMA,
            pltpu.SemaphoreType.DMA,
            pltpu.SemaphoreType.DMA,
            pltpu.SemaphoreType.DMA,
        ],
    )
    out = f(table, tok, pe)
    return out.reshape(B, S, D)


# speedup vs baseline: 1.7161x; 1.5215x over previous
import functools

import numpy as np

import jax
import jax.numpy as jnp
from jax import lax
from jax.experimental import pallas as pl
from jax.experimental.pallas import tpu as pltpu
from jax.experimental.pallas import tpu_sc as plsc

VOCAB = 100000
D = 768
B = 4
S = 2048
N = B * S
NC = 2
NS = 16
NW = NC * NS
TPW = N // NW
CH = 32
NCH = TPW // CH
LANES = 16
VEC = D // LANES


@functools.lru_cache(maxsize=1)
def _positional_encoding(max_seq, d_model):
    pos = np.arange(max_seq, dtype=np.float32)[:, None]
    i = np.arange(0, d_model, 2, dtype=np.float32)[None, :]
    denom = np.power(np.float32(10000.0), i / np.float32(d_model))
    arg = (pos / denom).astype(np.float32)
    pe = np.stack([np.sin(arg), np.cos(arg)], axis=2).astype(np.float32)
    return jnp.asarray(pe.reshape(max_seq, d_model))


def _body(table, tokens, pe, out, idx_v,
          rows0, rows1, pe0, pe1,
          sg0, sg1, sp0, sp1):
    rows = (rows0, rows1)
    pes = (pe0, pe1)
    sgs = (sg0, sg1)
    sps = (sp0, sp1)
    wid = lax.axis_index("s") * NC + lax.axis_index("c")
    base = wid * TPW
    pltpu.sync_copy(tokens.at[pl.ds(base, TPW)], idx_v)
    s0 = lax.rem(base, S)

    def start(c):
        i = c % 2
        cb = c * CH
        pltpu.async_copy(table.at[idx_v.at[pl.ds(cb, CH)]], rows[i], sgs[i])
        pltpu.async_copy(pe.at[pl.ds(s0 + cb, CH)], pes[i], sps[i])

    start(0)
    start(1)
    for c in range(NCH):
        i = c % 2
        cb = c * CH
        pltpu.make_async_copy(table.at[idx_v.at[pl.ds(cb, CH)]],
                              rows[i], sgs[i]).wait()
        pltpu.make_async_copy(pe.at[pl.ds(s0 + cb, CH)], pes[i],
                              sps[i]).wait()

        def add_row(t, carry):
            for j in range(VEC):
                sl = (t, pl.ds(j * LANES, LANES))
                rows[i][sl] = rows[i][sl] + pes[i][sl]
            return carry

        lax.fori_loop(0, CH, add_row, 0)
        pltpu.sync_copy(rows[i], out.at[pl.ds(base + cb, CH)])
        if c + 2 < NCH:
            start(c + 2)


@jax.jit
def kernel(tokens, table):
    pe = _positional_encoding(S, D)
    tok = tokens.reshape(N).astype(jnp.int32)
    mesh = plsc.VectorSubcoreMesh(core_axis_name="c", subcore_axis_name="s")
    f = pl.kernel(
        _body,
        out_type=jax.ShapeDtypeStruct((N, D), jnp.float32),
        mesh=mesh,
        scratch_types=[
            pltpu.VMEM((TPW,), jnp.int32),
            pltpu.VMEM((CH, D), jnp.float32),
            pltpu.VMEM((CH, D), jnp.float32),
            pltpu.VMEM((CH, D), jnp.float32),
            pltpu.VMEM((CH, D), jnp.float32),
            pltpu.SemaphoreType.DMA,
            pltpu.SemaphoreType.DMA,
            pltpu.SemaphoreType.DMA,
            pltpu.SemaphoreType.DMA,
        ],
    )
    out = f(table, tok, pe)
    return out.reshape(B, S, D)
